# Initial kernel scaffold; baseline (speedup 1.0000x reference)
import functools

import jax
import jax.numpy as jnp
from jax import lax
from jax.experimental import pallas as pl
from jax.experimental.pallas import tpu as pltpu
from jax.experimental.pallas import tpu_sc as plsc

N = 10000
D = 128
E = 320000
NC = 2
NS = 16
L = 16
DW = 16
K = 80
EPT = E // NS
NCHUNK = EPT // K
RPT = N // NS
MB = 2000


def _mm_body(x_ref, w_ref, o_ref):
    o_ref[...] = jnp.dot(x_ref[0], w_ref[0],
                         preferred_element_type=jnp.float32)[None]


def _premultiply(xs, Ws):
    return pl.pallas_call(
        _mm_body,
        grid=(2, N // MB),
        in_specs=[
            pl.BlockSpec((1, MB, D), lambda i, j: (i, j, 0)),
            pl.BlockSpec((1, D, D), lambda i, j: (i, 0, 0)),
        ],
        out_specs=pl.BlockSpec((1, MB, D), lambda i, j: (i, j, 0)),
        out_shape=jax.ShapeDtypeStruct((2, N, D), jnp.float32),
    )(xs, Ws)


def _sc_body(table_hbm, src_hbm, dst_hbm, bias_hbm, z128_hbm, z16_hbm,
             out_hbm,
             sidx, didx, fbuf, ones_v, obuf, dbuf, bbuf, acc, deg, gsem):
    c = lax.axis_index("c")
    t = lax.axis_index("s")
    r0 = t * RPT

    pltpu.sync_copy(z128_hbm.at[pl.ds(r0, RPT)], acc.at[pl.ds(r0, RPT)])
    pltpu.sync_copy(z16_hbm.at[pl.ds(r0, RPT)], deg.at[pl.ds(r0, RPT)])

    def _init_ones(r, _):
        ones_v[r, :] = jnp.ones((L,), jnp.float32)
        return 0
    lax.fori_loop(0, K, _init_ones, 0)

    plsc.subcore_barrier()

    def _chunk(j, _):
        off = t * EPT + j * K
        pltpu.sync_copy(src_hbm.at[c, pl.ds(off, K)], sidx)
        pltpu.sync_copy(dst_hbm.at[c, pl.ds(off, K)], didx)
        pltpu.async_copy(table_hbm.at[sidx], fbuf, gsem).wait()
        pltpu.sync_copy(fbuf, acc.at[didx], add=True)
        pltpu.sync_copy(ones_v, deg.at[didx], add=True)
        return 0
    lax.fori_loop(0, NCHUNK, _chunk, 0)

    plsc.subcore_barrier()

    pltpu.sync_copy(acc.at[pl.ds(r0, RPT)], obuf)
    pltpu.sync_copy(deg.at[pl.ds(r0, RPT)], dbuf)
    pltpu.sync_copy(bias_hbm.at[c], bbuf)

    def _finish(r, _):
        dvec = dbuf[r, :]
        inv = 1.0 / jnp.maximum(dvec, 1.0)
        for f in range(D // L):
            sl = pl.ds(f * L, L)
            obuf[r, sl] = obuf[r, sl] * inv + bbuf[sl]
        return 0
    lax.fori_loop(0, RPT, _finish, 0)

    pltpu.sync_copy(obuf, out_hbm.at[c, pl.ds(r0, RPT)])


def _sc_conv(table, src, dst, bias, z128, z16):
    mesh = plsc.VectorSubcoreMesh(core_axis_name="c", subcore_axis_name="s",
                                  num_cores=NC, num_subcores=NS)
    return pl.kernel(
        _sc_body,
        out_type=jax.ShapeDtypeStruct((NC, N, D), jnp.float32),
        mesh=mesh,
        scratch_types=[
            pltpu.VMEM((K,), jnp.int32),
            pltpu.VMEM((K,), jnp.int32),
            pltpu.VMEM((K, D), jnp.float32),
            pltpu.VMEM((K, DW), jnp.float32),
            pltpu.VMEM((RPT, D), jnp.float32),
            pltpu.VMEM((RPT, DW), jnp.float32),
            pltpu.VMEM((D,), jnp.float32),
            pltpu.VMEM_SHARED((N, D), jnp.float32),
            pltpu.VMEM_SHARED((N, DW), jnp.float32),
            pltpu.SemaphoreType.DMA,
        ],
    )(table, src, dst, bias, z128, z16)


def kernel(x_user, x_item, edge_index_rates, edge_index_rev,
           W_rates, b_rates, W_rev, b_rev):
    xs = jnp.stack([x_item, x_user])
    Ws = jnp.stack([W_rev, W_rates])
    table = _premultiply(xs, Ws).reshape(NC * N, D)

    src = jnp.stack([
        edge_index_rev[0].astype(jnp.int32),
        edge_index_rates[0].astype(jnp.int32) + N,
    ])
    dst = jnp.stack([
        edge_index_rev[1].astype(jnp.int32),
        edge_index_rates[1].astype(jnp.int32),
    ])
    bias = jnp.stack([b_rev, b_rates])
    z128 = jnp.zeros((N, D), jnp.float32)
    z16 = jnp.zeros((N, DW), jnp.float32)

    return _sc_conv(table, src, dst, bias, z128, z16)


# speedup vs baseline: 4.5548x; 4.5548x over previous
import jax
import jax.numpy as jnp
from jax import lax
from jax.experimental import pallas as pl
from jax.experimental.pallas import tpu as pltpu
from jax.experimental.pallas import tpu_sc as plsc

N = 10000
ND = 10016
D = 128
DH = 64
E = 320000
NC = 2
NS = 16
L = 16
K = 80
EPT = E // NS
NBLK = 5
BCH = EPT // K // NBLK
RPT = N // NS
RD = 640
RBF = 125
MB = 2000


def _mm_body(x_ref, w_ref, o_ref):
    o_ref[...] = jnp.dot(x_ref[0], w_ref[0],
                         preferred_element_type=jnp.float32)[None]


def _premultiply(xs, Ws):
    return pl.pallas_call(
        _mm_body,
        grid=(2, N // MB),
        in_specs=[
            pl.BlockSpec((1, MB, D), lambda i, j: (i, j, 0)),
            pl.BlockSpec((1, D, D), lambda i, j: (i, 0, 0)),
        ],
        out_specs=pl.BlockSpec((1, MB, D), lambda i, j: (i, j, 0)),
        out_shape=jax.ShapeDtypeStruct((2, N, D), jnp.float32),
    )(xs, Ws)


def _sc_body(table_hbm, src_hbm, dst_hbm, bias_hbm, zacc_hbm,
             out_hbm,
             sidx_blk, didx_blk, fbuf, obuf, dsum, ones_v, bbuf,
             acc, deg, gsem):
    c = lax.axis_index("c")
    t = lax.axis_index("s")
    r0 = t * RPT
    s0 = lax.rem(r0, 8)
    a0 = pl.multiple_of(r0 - s0, 8)

    ones16 = jnp.ones((L,), jnp.float32)
    zeros16 = jnp.zeros((L,), jnp.float32)
    coff = jnp.full((L,), c * 2 * N, jnp.int32)

    def _fill(ref, n, vec):
        def body(i, _):
            ref[pl.ds(i * L, L)] = vec
            return 0
        lax.fori_loop(0, n // L, body, 0)

    _fill(ones_v, K, ones16)

    for rel in range(2):
        pltpu.sync_copy(zacc_hbm, acc.at[pl.ds(r0, RPT)])
        _fill(dsum, RD, zeros16)
        pltpu.sync_copy(dsum, deg.at[pl.ds(a0, RD)])

        plsc.subcore_barrier()

        rb = rel * (E // K) + t * (EPT // K)

        def _block(b, _):
            pltpu.sync_copy(src_hbm.at[pl.ds(rb + b * BCH, BCH)], sidx_blk)
            pltpu.sync_copy(dst_hbm.at[pl.ds(rb + b * BCH, BCH)], didx_blk)

            def _shift(r, _):
                for u in range(K // L):
                    sl = pl.ds(u * L, L)
                    sidx_blk[r, sl] = sidx_blk[r, sl] + coff
                return 0
            lax.fori_loop(0, BCH, _shift, 0)

            def _chunk(j, _):
                pltpu.async_copy(table_hbm.at[sidx_blk.at[j]], fbuf,
                                 gsem).wait()
                pltpu.sync_copy(fbuf, acc.at[didx_blk.at[j]], add=True)
                pltpu.sync_copy(ones_v, deg.at[didx_blk.at[j]], add=True)
                return 0
            lax.fori_loop(0, BCH, _chunk, 0)
            return 0
        lax.fori_loop(0, NBLK, _block, 0)

        plsc.subcore_barrier()

        pltpu.sync_copy(deg.at[pl.ds(a0, RD)], dsum)
        pltpu.sync_copy(bias_hbm.at[pl.ds(rel * D + c * DH, DH)], bbuf)

        for q in range(RPT // RBF):
            pltpu.sync_copy(acc.at[pl.ds(r0 + q * RBF, RBF)], obuf)

            def _finish(r, _):
                ridx = jnp.full((L,), s0 + q * RBF, jnp.int32) + \
                    jnp.full((L,), r, jnp.int32)
                dvec = plsc.load_gather(dsum, [ridx])
                inv = 1.0 / jnp.maximum(dvec, 1.0)
                for f in range(DH // L):
                    sl = pl.ds(f * L, L)
                    obuf[r, sl] = obuf[r, sl] * inv + bbuf[sl]
                return 0
            lax.fori_loop(0, RBF, _finish, 0)

            pltpu.sync_copy(obuf,
                            out_hbm.at[rel, c, pl.ds(r0 + q * RBF, RBF)])

        plsc.subcore_barrier()


def _sc_conv(table, src, dst, bias, zacc):
    mesh = plsc.VectorSubcoreMesh(core_axis_name="c", subcore_axis_name="s",
                                  num_cores=NC, num_subcores=NS)
    return pl.kernel(
        _sc_body,
        out_type=jax.ShapeDtypeStruct((2, NC, N, DH), jnp.float32),
        mesh=mesh,
        compiler_params=pltpu.CompilerParams(use_tc_tiling_on_sc=False,
                                             needs_layout_passes=False),
        scratch_types=[
            pltpu.VMEM((BCH, K), jnp.int32),
            pltpu.VMEM((BCH, K), jnp.int32),
            pltpu.VMEM((K, DH), jnp.float32),
            pltpu.VMEM((RBF, DH), jnp.float32),
            pltpu.VMEM((RD,), jnp.float32),
            pltpu.VMEM((K,), jnp.float32),
            pltpu.VMEM((DH,), jnp.float32),
            pltpu.VMEM_SHARED((N, DH), jnp.float32),
            pltpu.VMEM_SHARED((ND,), jnp.float32),
            pltpu.SemaphoreType.DMA,
        ],
    )(table, src, dst, bias, zacc)


def kernel(x_user, x_item, edge_index_rates, edge_index_rev,
           W_rates, b_rates, W_rev, b_rev):
    xs = jnp.stack([x_item, x_user])
    Ws = jnp.stack([W_rev, W_rates])
    y = _premultiply(xs, Ws).reshape(NC * N, D)

    table = jnp.concatenate([y[:, :DH], y[:, DH:]], axis=0)

    src = jnp.concatenate([
        edge_index_rev[0].astype(jnp.int32),
        edge_index_rates[0].astype(jnp.int32) + N,
    ]).reshape(2 * E // K, K)
    dst = jnp.concatenate([
        edge_index_rev[1].astype(jnp.int32),
        edge_index_rates[1].astype(jnp.int32),
    ]).reshape(2 * E // K, K)
    bias = jnp.concatenate([b_rev, b_rates])
    zacc = jnp.zeros((RPT, DH), jnp.float32)

    out = _sc_conv(table, src, dst, bias, zacc)
    return jnp.concatenate([out[:, 0], out[:, 1]], axis=-1)
